# baseline (speedup 1.0000x reference)
import functools

import jax
import jax.numpy as jnp
from jax import lax
from jax.experimental import pallas as pl
from jax.experimental.pallas import tpu as pltpu
from jax.experimental.pallas import tpu_sc as plsc

B = 4096
NUM_ENTITIES = 1024
NUM_RELATIONS = 32

_INFO = plsc.get_sparse_core_info()
_NC = _INFO.num_cores
_NS = _INFO.num_subcores
_L = _INFO.num_lanes
_NW = _NC * _NS
_BPW = B // _NW


def _dkb_body(flat_hbm, subj_hbm, rel_hbm, obj_hbm, out_hbm,
              subj_v, rel_v, obj_v, idx_v, val_v, sem):
    wid = lax.axis_index("s") * _NC + lax.axis_index("c")
    base = wid * _BPW
    pltpu.sync_copy(subj_hbm.at[pl.ds(base, _BPW)], subj_v)
    pltpu.sync_copy(rel_hbm.at[pl.ds(base, _BPW)], rel_v)
    pltpu.sync_copy(obj_hbm.at[pl.ds(base, _BPW)], obj_v)
    for i in range(_BPW // _L):
        sl = pl.ds(i * _L, _L)
        idx_v[sl] = (rel_v[sl] * (NUM_ENTITIES * NUM_ENTITIES)
                     + subj_v[sl] * NUM_ENTITIES
                     + obj_v[sl])
    pltpu.async_copy(flat_hbm.at[idx_v], val_v, sem).wait()
    for i in range(_BPW // _L):
        sl = pl.ds(i * _L, _L)
        x = val_v[sl]
        val_v[sl] = 1.0 / (1.0 + jnp.exp(-x))
    pltpu.sync_copy(val_v, out_hbm.at[pl.ds(base, _BPW)])


@jax.jit
def kernel(subject_ids, relation_ids, object_ids, truth_values):
    flat = truth_values.reshape(-1)
    run = functools.partial(
        pl.kernel,
        out_type=jax.ShapeDtypeStruct((B,), jnp.float32),
        mesh=plsc.VectorSubcoreMesh(core_axis_name="c", subcore_axis_name="s"),
        scratch_types=[
            pltpu.VMEM((_BPW,), jnp.int32),
            pltpu.VMEM((_BPW,), jnp.int32),
            pltpu.VMEM((_BPW,), jnp.int32),
            pltpu.VMEM((_BPW,), jnp.int32),
            pltpu.VMEM((_BPW,), jnp.float32),
            pltpu.SemaphoreType.DMA,
        ],
    )(_dkb_body)
    return run(flat,
               subject_ids.astype(jnp.int32),
               relation_ids.astype(jnp.int32),
               object_ids.astype(jnp.int32))


# speedup vs baseline: 5.3451x; 5.3451x over previous
import functools

import jax
import jax.numpy as jnp
from jax import lax
from jax.experimental import pallas as pl
from jax.experimental.pallas import tpu as pltpu
from jax.experimental.pallas import tpu_sc as plsc

B = 4096
NUM_ENTITIES = 1024
NUM_RELATIONS = 32

_INFO = plsc.get_sparse_core_info()
_NC = _INFO.num_cores
_NS = _INFO.num_subcores
_L = _INFO.num_lanes
_NW = _NC * _NS
_BPW = B // _NW

_S = 8
_LOG2_S = _S.bit_length() - 1


def _dkb_body(flat_hbm, subj_hbm, rel_hbm, obj_hbm, out_hbm,
              subj_v, rel_v, obj_v, idx_v, val_v, sem):
    wid = lax.axis_index("s") * _NC + lax.axis_index("c")
    base = wid * _BPW
    pltpu.sync_copy(subj_hbm.at[pl.ds(base, _BPW)], subj_v)
    pltpu.sync_copy(rel_hbm.at[pl.ds(base, _BPW)], rel_v)
    pltpu.sync_copy(obj_hbm.at[pl.ds(base, _BPW)], obj_v)
    for i in range(_BPW // _L):
        sl = pl.ds(i * _L, _L)
        s = subj_v[sl]
        o = obj_v[sl]
        idx_v[sl] = (rel_v[sl] * (NUM_ENTITIES * NUM_ENTITIES)
                     + lax.shift_right_logical(s, _LOG2_S) * (NUM_ENTITIES * _S)
                     + lax.shift_right_logical(o, 7) * (128 * _S)
                     + (s & (_S - 1)) * 128
                     + (o & 127))
    pltpu.async_copy(flat_hbm.at[idx_v], val_v, sem).wait()
    for i in range(_BPW // _L):
        sl = pl.ds(i * _L, _L)
        x = val_v[sl]
        val_v[sl] = 1.0 / (1.0 + jnp.exp(-x))
    pltpu.sync_copy(val_v, out_hbm.at[pl.ds(base, _BPW)])


@jax.jit
def kernel(subject_ids, relation_ids, object_ids, truth_values):
    flat = (truth_values
            .reshape(NUM_RELATIONS, NUM_ENTITIES // _S, _S, NUM_ENTITIES // 128, 128)
            .transpose(0, 1, 3, 2, 4)
            .reshape(-1))
    run = functools.partial(
        pl.kernel,
        out_type=jax.ShapeDtypeStruct((B,), jnp.float32),
        mesh=plsc.VectorSubcoreMesh(core_axis_name="c", subcore_axis_name="s"),
        scratch_types=[
            pltpu.VMEM((_BPW,), jnp.int32),
            pltpu.VMEM((_BPW,), jnp.int32),
            pltpu.VMEM((_BPW,), jnp.int32),
            pltpu.VMEM((_BPW,), jnp.int32),
            pltpu.VMEM((_BPW,), jnp.float32),
            pltpu.SemaphoreType.DMA,
        ],
    )(_dkb_body)
    return run(flat,
               subject_ids.astype(jnp.int32),
               relation_ids.astype(jnp.int32),
               object_ids.astype(jnp.int32))


# speedup vs baseline: 5.5492x; 1.0382x over previous
import functools

import jax
import jax.numpy as jnp
from jax import lax
from jax.experimental import pallas as pl
from jax.experimental.pallas import tpu as pltpu
from jax.experimental.pallas import tpu_sc as plsc

B = 4096
NUM_ENTITIES = 1024
NUM_RELATIONS = 32

_INFO = plsc.get_sparse_core_info()
_NC = _INFO.num_cores
_NS = _INFO.num_subcores
_L = _INFO.num_lanes
_NW = _NC * _NS
_BPW = B // _NW

_S = 8
_LOG2_S = _S.bit_length() - 1


def _dkb_body(flat_hbm, subj_hbm, rel_hbm, obj_hbm, out_hbm,
              subj_v, rel_v, obj_v, idx_v, val_v, sem):
    wid = lax.axis_index("s") * _NC + lax.axis_index("c")
    base = wid * _BPW
    c1 = pltpu.async_copy(subj_hbm.at[pl.ds(base, _BPW)], subj_v, sem)
    c2 = pltpu.async_copy(rel_hbm.at[pl.ds(base, _BPW)], rel_v, sem)
    c3 = pltpu.async_copy(obj_hbm.at[pl.ds(base, _BPW)], obj_v, sem)
    c1.wait()
    c2.wait()
    c3.wait()
    for i in range(_BPW // _L):
        sl = pl.ds(i * _L, _L)
        s = subj_v[sl]
        o = obj_v[sl]
        idx_v[sl] = (rel_v[sl] * (NUM_ENTITIES * NUM_ENTITIES)
                     + lax.shift_right_logical(s, _LOG2_S) * (NUM_ENTITIES * _S)
                     + lax.shift_right_logical(o, 7) * (128 * _S)
                     + (s & (_S - 1)) * 128
                     + (o & 127))
    pltpu.async_copy(flat_hbm.at[idx_v], val_v, sem).wait()
    for i in range(_BPW // _L):
        sl = pl.ds(i * _L, _L)
        x = val_v[sl]
        val_v[sl] = 1.0 / (1.0 + jnp.exp(-x))
    pltpu.sync_copy(val_v, out_hbm.at[pl.ds(base, _BPW)])


@jax.jit
def kernel(subject_ids, relation_ids, object_ids, truth_values):
    flat = (truth_values
            .reshape(NUM_RELATIONS, NUM_ENTITIES // _S, _S, NUM_ENTITIES // 128, 128)
            .transpose(0, 1, 3, 2, 4)
            .reshape(-1))
    run = functools.partial(
        pl.kernel,
        out_type=jax.ShapeDtypeStruct((B,), jnp.float32),
        mesh=plsc.VectorSubcoreMesh(core_axis_name="c", subcore_axis_name="s"),
        scratch_types=[
            pltpu.VMEM((_BPW,), jnp.int32),
            pltpu.VMEM((_BPW,), jnp.int32),
            pltpu.VMEM((_BPW,), jnp.int32),
            pltpu.VMEM((_BPW,), jnp.int32),
            pltpu.VMEM((_BPW,), jnp.float32),
            pltpu.SemaphoreType.DMA,
        ],
    )(_dkb_body)
    return run(flat,
               subject_ids.astype(jnp.int32),
               relation_ids.astype(jnp.int32),
               object_ids.astype(jnp.int32))


# speedup vs baseline: 5.6205x; 1.0129x over previous
import functools

import jax
import jax.numpy as jnp
from jax import lax
from jax.experimental import pallas as pl
from jax.experimental.pallas import tpu as pltpu
from jax.experimental.pallas import tpu_sc as plsc

B = 4096
NUM_ENTITIES = 1024
NUM_RELATIONS = 32

_INFO = plsc.get_sparse_core_info()
_NC = _INFO.num_cores
_NS = _INFO.num_subcores
_L = _INFO.num_lanes
_NW = _NC * _NS
_BPW = B // _NW

_S = 8
_LOG2_S = _S.bit_length() - 1


def _dkb_body(flat_hbm, subj_hbm, rel_hbm, obj_hbm, out_hbm,
              subj_v, rel_v, obj_v, idx_v, val_v, sem, gsems, osems):
    wid = lax.axis_index("s") * _NC + lax.axis_index("c")
    base = wid * _BPW
    c1 = pltpu.async_copy(subj_hbm.at[pl.ds(base, _BPW)], subj_v, sem)
    c2 = pltpu.async_copy(rel_hbm.at[pl.ds(base, _BPW)], rel_v, sem)
    c3 = pltpu.async_copy(obj_hbm.at[pl.ds(base, _BPW)], obj_v, sem)
    c1.wait()
    c2.wait()
    c3.wait()
    half = _BPW // 2
    gathers = []
    for h in range(2):
        for i in range(h * half // _L, (h + 1) * half // _L):
            sl = pl.ds(i * _L, _L)
            s = subj_v[sl]
            o = obj_v[sl]
            idx_v[sl] = (rel_v[sl] * (NUM_ENTITIES * NUM_ENTITIES)
                         + lax.shift_right_logical(s, _LOG2_S) * (NUM_ENTITIES * _S)
                         + lax.shift_right_logical(o, 7) * (128 * _S)
                         + (s & (_S - 1)) * 128
                         + (o & 127))
        gathers.append(pltpu.async_copy(
            flat_hbm.at[idx_v.at[pl.ds(h * half, half)]],
            val_v.at[pl.ds(h * half, half)], gsems.at[h]))
    outs = []
    for h in range(2):
        gathers[h].wait()
        for i in range(h * half // _L, (h + 1) * half // _L):
            sl = pl.ds(i * _L, _L)
            x = val_v[sl]
            val_v[sl] = 1.0 / (1.0 + jnp.exp(-x))
        outs.append(pltpu.async_copy(
            val_v.at[pl.ds(h * half, half)],
            out_hbm.at[pl.ds(base + h * half, half)], osems.at[h]))
    for h in range(2):
        outs[h].wait()


@jax.jit
def kernel(subject_ids, relation_ids, object_ids, truth_values):
    flat = (truth_values
            .reshape(NUM_RELATIONS, NUM_ENTITIES // _S, _S, NUM_ENTITIES // 128, 128)
            .transpose(0, 1, 3, 2, 4)
            .reshape(-1))
    run = functools.partial(
        pl.kernel,
        out_type=jax.ShapeDtypeStruct((B,), jnp.float32),
        mesh=plsc.VectorSubcoreMesh(core_axis_name="c", subcore_axis_name="s"),
        scratch_types=[
            pltpu.VMEM((_BPW,), jnp.int32),
            pltpu.VMEM((_BPW,), jnp.int32),
            pltpu.VMEM((_BPW,), jnp.int32),
            pltpu.VMEM((_BPW,), jnp.int32),
            pltpu.VMEM((_BPW,), jnp.float32),
            pltpu.SemaphoreType.DMA,
            pltpu.SemaphoreType.DMA((2,)),
            pltpu.SemaphoreType.DMA((2,)),
        ],
    )(_dkb_body)
    return run(flat,
               subject_ids.astype(jnp.int32),
               relation_ids.astype(jnp.int32),
               object_ids.astype(jnp.int32))
